# baseline (speedup 1.0000x reference)
import functools

import jax
import jax.numpy as jnp
from jax import lax
from jax.experimental import pallas as pl
from jax.experimental.pallas import tpu as pltpu
from jax.experimental.pallas import tpu_sc as plsc

VOCAB = 100000
D = 64
B, S = 1024, 200
NW = 32
PAIR_SEQ = B // 16
Q0 = 18
Q1 = PAIR_SEQ - Q0
QMAX = max(Q0, Q1)
HALF = 100
NBUF = 6

_mesh = plsc.VectorSubcoreMesh(core_axis_name="c", subcore_axis_name="s")


@functools.partial(
    pl.kernel,
    mesh=_mesh,
    out_type=jax.ShapeDtypeStruct((B, S, D), jnp.float32),
    scratch_types=[
        pltpu.VMEM((QMAX * 2, HALF), jnp.int32),
        pltpu.VMEM((QMAX * S + 16,), jnp.int32),
        pltpu.VMEM((S, D), jnp.float32),
        pltpu.VMEM((S, D), jnp.float32),
        pltpu.VMEM((S, D), jnp.float32),
        pltpu.VMEM((S, D), jnp.float32),
        pltpu.VMEM((S, D), jnp.float32),
        pltpu.VMEM((S, D), jnp.float32),
        pltpu.VMEM((S, D), jnp.float32),
        pltpu.SemaphoreType.DMA,
        pltpu.SemaphoreType.DMA,
        pltpu.SemaphoreType.DMA,
        pltpu.SemaphoreType.DMA,
        pltpu.SemaphoreType.DMA,
        pltpu.SemaphoreType.DMA,
        pltpu.SemaphoreType.DMA,
        pltpu.SemaphoreType.DMA,
        pltpu.SemaphoreType.DMA,
        pltpu.SemaphoreType.DMA,
        pltpu.SemaphoreType.DMA,
        pltpu.SemaphoreType.DMA,
    ],
    compiler_params=pltpu.CompilerParams(use_tc_tiling_on_sc=False),
)
def _encoder(ids_stream, ids_chk, table, pos, out,
             idx_v, chk_v, pos_v, b0, b1, b2, b3, b4, b5,
             g0, g1, g2, g3, g4, g5, s0, s1, s2, s3, s4, s5):
    bufs = (b0, b1, b2, b3, b4, b5)
    gsems = (g0, g1, g2, g3, g4, g5)
    ssems = (s0, s1, s2, s3, s4, s5)

    cid = lax.axis_index("c")
    sid = lax.axis_index("s")
    base = sid * PAIR_SEQ + lax.mul(cid, Q0)
    pltpu.sync_copy(ids_stream.at[pl.ds(2 * base, 2 * QMAX)], idx_v)
    pltpu.sync_copy(ids_chk.at[pl.ds(base * S, QMAX * S)],
                    chk_v.at[pl.ds(0, QMAX * S)])
    pltpu.sync_copy(pos.at[pl.ds(0, S)], pos_v)

    def gathers(slot, c):
        return (pltpu.make_async_copy(table.at[idx_v.at[2 * c]],
                                      bufs[slot].at[pl.ds(0, HALF)],
                                      gsems[slot]),
                pltpu.make_async_copy(table.at[idx_v.at[2 * c + 1]],
                                      bufs[slot].at[pl.ds(HALF, HALF)],
                                      gsems[slot]))

    def store(slot, c):
        return pltpu.make_async_copy(bufs[slot], out.at[base + c], ssems[slot])

    def compute(slot, c, has_pad):
        buf = bufs[slot]
        tok0 = c * S

        @pl.when(has_pad)
        def _():
            def _fix(r, c2):
                idv = chk_v[pl.ds(tok0 + r, 16)]
                @pl.when(idv[0] == 0)
                def _():
                    zero = jnp.zeros((16,), jnp.float32)
                    for k in range(4):
                        buf[r, pl.ds(k * 16, 16)] = zero
                return c2
            lax.fori_loop(0, S, _fix, 0)

        def _add(r, c2):
            for k in range(4):
                sl = pl.ds(k * 16, 16)
                buf[r, sl] = buf[r, sl] + pos_v[r, sl]
            return c2
        lax.fori_loop(0, S, _add, 0)

    def run(n):
        def _mn(i, acc):
            return jnp.minimum(acc, chk_v[pl.ds(i * 16, 16)])

        acc = lax.fori_loop(0, n * S // 16, _mn,
                            jnp.full((16,), jnp.iinfo(jnp.int32).max, jnp.int32))
        lanes = lax.iota(jnp.int32, 16)
        for shift in (8, 4, 2, 1):
            g = lax.gather(
                acc, lax.rem(lanes + shift, 16)[:, None],
                dimension_numbers=lax.GatherDimensionNumbers(
                    offset_dims=(), collapsed_slice_dims=(0,),
                    start_index_map=(0,)),
                slice_sizes=(1,), mode=lax.GatherScatterMode.PROMISE_IN_BOUNDS)
            acc = jnp.minimum(acc, g)
        has_pad = acc[0] == 0

        for c0 in range(NBUF - 1):
            a, b = gathers(c0, jnp.int32(c0))
            a.start()
            b.start()

        iters = n // NBUF

        def _iter(i, carry):
            for j in range(NBUF):
                c = NBUF * i + j
                nxt = c + NBUF - 1
                tgt = (j + NBUF - 1) % NBUF

                def _prefetch():
                    a, b = gathers(tgt, nxt)
                    a.start()
                    b.start()

                if j == 0:
                    @pl.when(i > 0)
                    def _():
                        store(tgt, nxt - NBUF).wait()
                    _prefetch()
                else:
                    @pl.when(i < iters - 1)
                    def _():
                        store(tgt, nxt - NBUF).wait()
                        _prefetch()

                ga, gb = gathers(j, c)
                ga.wait()
                gb.wait()
                compute(j, c, has_pad)
                store(j, c).start()
            return carry

        lax.fori_loop(0, iters, _iter, 0)

        for j in range(NBUF):
            store(j, jnp.int32(n - NBUF + j)).wait()

    @pl.when(cid == 0)
    def _():
        run(Q0)

    @pl.when(cid == 1)
    def _():
        run(Q1)


def kernel(input_ids, token_table, pos_table):
    ids = input_ids.astype(jnp.int32)
    ids_stream = ids.reshape(B * 2, HALF)
    ids_chk = ids.reshape(B * S)
    return _encoder(ids_stream, ids_chk, token_table, pos_table)


# speedup vs baseline: 1.0331x; 1.0331x over previous
import functools

import jax
import jax.numpy as jnp
from jax import lax
from jax.experimental import pallas as pl
from jax.experimental.pallas import tpu as pltpu
from jax.experimental.pallas import tpu_sc as plsc

VOCAB = 100000
D = 64
B, S = 1024, 200
NW = 32
PAIR_SEQ = B // 16
Q0 = 16
Q1 = PAIR_SEQ - Q0
QMAX = max(Q0, Q1)
HALF = 100
NBUF = 4

_mesh = plsc.VectorSubcoreMesh(core_axis_name="c", subcore_axis_name="s")


@functools.partial(
    pl.kernel,
    mesh=_mesh,
    out_type=jax.ShapeDtypeStruct((B, S, D), jnp.float32),
    scratch_types=[
        pltpu.VMEM((QMAX * 2, HALF), jnp.int32),
        pltpu.VMEM((QMAX * S + 16,), jnp.int32),
        pltpu.VMEM((S, D), jnp.float32),
        pltpu.VMEM((S, D), jnp.float32),
        pltpu.VMEM((S, D), jnp.float32),
        pltpu.VMEM((S, D), jnp.float32),
        pltpu.VMEM((S, D), jnp.float32),
        pltpu.SemaphoreType.DMA,
        pltpu.SemaphoreType.DMA,
        pltpu.SemaphoreType.DMA,
        pltpu.SemaphoreType.DMA,
        pltpu.SemaphoreType.DMA,
        pltpu.SemaphoreType.DMA,
        pltpu.SemaphoreType.DMA,
        pltpu.SemaphoreType.DMA,
    ],
    compiler_params=pltpu.CompilerParams(use_tc_tiling_on_sc=False),
)
def _encoder(ids_stream, ids_chk, table, pos, out,
             idx_v, chk_v, pos_v, b0, b1, b2, b3,
             g0, g1, g2, g3, s0, s1, s2, s3):
    bufs = (b0, b1, b2, b3)
    gsems = (g0, g1, g2, g3)
    ssems = (s0, s1, s2, s3)

    cid = lax.axis_index("c")
    sid = lax.axis_index("s")
    base = sid * PAIR_SEQ + lax.mul(cid, Q0)
    pltpu.sync_copy(ids_stream.at[pl.ds(2 * base, 2 * QMAX)], idx_v)
    pltpu.sync_copy(ids_chk.at[pl.ds(base * S, QMAX * S)],
                    chk_v.at[pl.ds(0, QMAX * S)])
    pltpu.sync_copy(pos.at[pl.ds(0, S)], pos_v)

    def gathers(slot, c):
        return (pltpu.make_async_copy(table.at[idx_v.at[2 * c]],
                                      bufs[slot].at[pl.ds(0, HALF)],
                                      gsems[slot]),
                pltpu.make_async_copy(table.at[idx_v.at[2 * c + 1]],
                                      bufs[slot].at[pl.ds(HALF, HALF)],
                                      gsems[slot]))

    def store(slot, c):
        return pltpu.make_async_copy(bufs[slot], out.at[base + c], ssems[slot])

    def compute(slot, c, has_pad):
        buf = bufs[slot]
        tok0 = c * S

        @pl.when(has_pad)
        def _():
            def _fix(r, c2):
                idv = chk_v[pl.ds(tok0 + r, 16)]
                @pl.when(idv[0] == 0)
                def _():
                    zero = jnp.zeros((16,), jnp.float32)
                    for k in range(4):
                        buf[r, pl.ds(k * 16, 16)] = zero
                return c2
            lax.fori_loop(0, S, _fix, 0)

        def _add(r, c2):
            for k in range(4):
                sl = pl.ds(k * 16, 16)
                buf[r, sl] = buf[r, sl] + pos_v[r, sl]
            return c2
        lax.fori_loop(0, S, _add, 0)

    def run(n):
        def _mn(i, acc):
            return jnp.minimum(acc, chk_v[pl.ds(i * 16, 16)])

        acc = lax.fori_loop(0, n * S // 16, _mn,
                            jnp.full((16,), jnp.iinfo(jnp.int32).max, jnp.int32))
        lanes = lax.iota(jnp.int32, 16)
        for shift in (8, 4, 2, 1):
            g = lax.gather(
                acc, lax.rem(lanes + shift, 16)[:, None],
                dimension_numbers=lax.GatherDimensionNumbers(
                    offset_dims=(), collapsed_slice_dims=(0,),
                    start_index_map=(0,)),
                slice_sizes=(1,), mode=lax.GatherScatterMode.PROMISE_IN_BOUNDS)
            acc = jnp.minimum(acc, g)
        has_pad = acc[0] == 0

        for c0 in range(NBUF - 1):
            a, b = gathers(c0, jnp.int32(c0))
            a.start()
            b.start()

        iters = n // NBUF

        def _iter(i, carry):
            for j in range(NBUF):
                c = NBUF * i + j
                nxt = c + NBUF - 1
                tgt = (j + NBUF - 1) % NBUF

                def _prefetch():
                    a, b = gathers(tgt, nxt)
                    a.start()
                    b.start()

                if j == 0:
                    @pl.when(i > 0)
                    def _():
                        store(tgt, nxt - NBUF).wait()
                    _prefetch()
                else:
                    @pl.when(i < iters - 1)
                    def _():
                        store(tgt, nxt - NBUF).wait()
                        _prefetch()

                ga, gb = gathers(j, c)
                ga.wait()
                gb.wait()
                compute(j, c, has_pad)
                store(j, c).start()
            return carry

        lax.fori_loop(0, iters, _iter, 0)

        for j in range(NBUF):
            store(j, jnp.int32(n - NBUF + j)).wait()

    @pl.when(cid == 0)
    def _():
        run(Q0)

    @pl.when(cid == 1)
    def _():
        run(Q1)


def kernel(input_ids, token_table, pos_table):
    ids = input_ids.astype(jnp.int32)
    ids_stream = ids.reshape(B * 2, HALF)
    ids_chk = ids.reshape(B * S)
    return _encoder(ids_stream, ids_chk, token_table, pos_table)
